# Initial kernel scaffold; baseline (speedup 1.0000x reference)
import jax
import jax.numpy as jnp
from jax.experimental import pallas as pl


def kernel(instruction_ids, embed_weight):
    raise NotImplementedError("write your pallas kernel here")


# speedup vs baseline: 2.8158x; 2.8158x over previous
import functools

import jax
import jax.numpy as jnp
from jax import lax
from jax.experimental import pallas as pl
from jax.experimental.pallas import tpu as pltpu
from jax.experimental.pallas import tpu_sc as plsc

B = 16384
L = 50
H = 32
NUM_CORES = 2
NUM_SUBCORES = 16
NW = NUM_CORES * NUM_SUBCORES
BPW = B // NW
CB = 64
NCHUNK = BPW // CB
IPC = CB * L
GSZ = 128
NG = IPC // GSZ
INV_L = 1.0 / L


def _embed_body(ids_hbm, table_hbm, out_hbm, idx_v, rows_v, out_v, sem):
    c = lax.axis_index("c")
    s = lax.axis_index("s")
    wid = s * NUM_CORES + c
    base = wid * BPW

    def chunk_body(g, carry):
        row0 = base + g * CB
        pltpu.sync_copy(ids_hbm.at[pl.ds(row0 * L, IPC)], idx_v)
        cps = [
            pltpu.async_copy(
                table_hbm.at[idx_v.at[pl.ds(j * GSZ, GSZ)]],
                rows_v.at[pl.ds(j * GSZ, GSZ)],
                sem,
            )
            for j in range(NG)
        ]
        for cp in cps:
            cp.wait()

        def row_body(r, carry2):
            off = r * L
            acc0 = jnp.zeros((16,), jnp.float32)
            acc1 = jnp.zeros((16,), jnp.float32)
            for j in range(L):
                acc0 = acc0 + rows_v[off + j, pl.ds(0, 16)]
                acc1 = acc1 + rows_v[off + j, pl.ds(16, 16)]
            out_v[r, pl.ds(0, 16)] = acc0 * INV_L
            out_v[r, pl.ds(16, 16)] = acc1 * INV_L
            return carry2

        lax.fori_loop(0, CB, row_body, 0)
        pltpu.sync_copy(out_v, out_hbm.at[pl.ds(row0, CB)])
        return carry

    lax.fori_loop(0, NCHUNK, chunk_body, 0)


@jax.jit
def _embed(ids_flat, table):
    mesh = plsc.VectorSubcoreMesh(
        core_axis_name="c",
        subcore_axis_name="s",
        num_cores=NUM_CORES,
        num_subcores=NUM_SUBCORES,
    )
    return pl.kernel(
        _embed_body,
        out_type=jax.ShapeDtypeStruct((B, H), jnp.float32),
        mesh=mesh,
        scratch_types=[
            pltpu.VMEM((IPC,), jnp.int32),
            pltpu.VMEM((IPC, H), jnp.float32),
            pltpu.VMEM((CB, H), jnp.float32),
            pltpu.SemaphoreType.DMA,
        ],
        compiler_params=pltpu.CompilerParams(use_tc_tiling_on_sc=False),
    )(ids_flat, table)


def kernel(instruction_ids, embed_weight):
    ids_flat = instruction_ids.astype(jnp.int32).reshape(-1)
    return _embed(ids_flat, embed_weight)


# speedup vs baseline: 2.9399x; 1.0441x over previous
import functools

import jax
import jax.numpy as jnp
from jax import lax
from jax.experimental import pallas as pl
from jax.experimental.pallas import tpu as pltpu
from jax.experimental.pallas import tpu_sc as plsc

B = 16384
L = 50
H = 32
NUM_CORES = 2
NUM_SUBCORES = 16
NW = NUM_CORES * NUM_SUBCORES
BPW = B // NW
CB = 32
NCHUNK = BPW // CB
NPAIR = NCHUNK // 2
IPC = CB * L
GSZ = 128
INV_L = 1.0 / L

_SPLITS = []
_off = 0
while _off < IPC:
    _n = min(GSZ, IPC - _off)
    _SPLITS.append((_off, _n))
    _off += _n


def _fire(table_hbm, idx_v, rows_v, sem):
    for off, n in _SPLITS:
        pltpu.async_copy(
            table_hbm.at[idx_v.at[pl.ds(off, n)]],
            rows_v.at[pl.ds(off, n)],
            sem,
        )


def _drain(table_hbm, idx_v, rows_v, sem):
    for off, n in _SPLITS:
        pltpu.make_async_copy(
            table_hbm.at[idx_v.at[pl.ds(off, n)]],
            rows_v.at[pl.ds(off, n)],
            sem,
        ).wait()


def _accum_store(rows_v, out_v, out_hbm, row0):
    def row_body(r, carry):
        off = r * L
        acc0 = jnp.zeros((16,), jnp.float32)
        acc1 = jnp.zeros((16,), jnp.float32)
        for j in range(L):
            acc0 = acc0 + rows_v[off + j, pl.ds(0, 16)]
            acc1 = acc1 + rows_v[off + j, pl.ds(16, 16)]
        out_v[r, pl.ds(0, 16)] = acc0 * INV_L
        out_v[r, pl.ds(16, 16)] = acc1 * INV_L
        return carry

    lax.fori_loop(0, CB, row_body, 0)
    pltpu.sync_copy(out_v, out_hbm.at[pl.ds(row0, CB)])


def _embed_body(ids_hbm, table_hbm, out_hbm,
                idx0, idx1, rows0, rows1, out_v, sem0, sem1):
    c = lax.axis_index("c")
    s = lax.axis_index("s")
    wid = s * NUM_CORES + c
    base = wid * BPW

    pltpu.sync_copy(ids_hbm.at[pl.ds(base * L, IPC)], idx0)
    _fire(table_hbm, idx0, rows0, sem0)

    def pair_body(i, carry):
        row_a = base + (2 * i) * CB
        row_b = row_a + CB
        pltpu.sync_copy(ids_hbm.at[pl.ds(row_b * L, IPC)], idx1)
        _fire(table_hbm, idx1, rows1, sem1)
        _drain(table_hbm, idx0, rows0, sem0)
        _accum_store(rows0, out_v, out_hbm, row_a)

        @pl.when(i < NPAIR - 1)
        def _():
            row_c = row_b + CB
            pltpu.sync_copy(ids_hbm.at[pl.ds(row_c * L, IPC)], idx0)
            _fire(table_hbm, idx0, rows0, sem0)

        _drain(table_hbm, idx1, rows1, sem1)
        _accum_store(rows1, out_v, out_hbm, row_b)
        return carry

    lax.fori_loop(0, NPAIR, pair_body, 0)


@jax.jit
def _embed(ids_flat, table):
    mesh = plsc.VectorSubcoreMesh(
        core_axis_name="c",
        subcore_axis_name="s",
        num_cores=NUM_CORES,
        num_subcores=NUM_SUBCORES,
    )
    return pl.kernel(
        _embed_body,
        out_type=jax.ShapeDtypeStruct((B, H), jnp.float32),
        mesh=mesh,
        scratch_types=[
            pltpu.VMEM((IPC,), jnp.int32),
            pltpu.VMEM((IPC,), jnp.int32),
            pltpu.VMEM((IPC, H), jnp.float32),
            pltpu.VMEM((IPC, H), jnp.float32),
            pltpu.VMEM((CB, H), jnp.float32),
            pltpu.SemaphoreType.DMA,
            pltpu.SemaphoreType.DMA,
        ],
        compiler_params=pltpu.CompilerParams(use_tc_tiling_on_sc=False),
    )(ids_flat, table)


def kernel(instruction_ids, embed_weight):
    ids_flat = instruction_ids.astype(jnp.int32).reshape(-1)
    return _embed(ids_flat, embed_weight)
